# Initial kernel scaffold; baseline (speedup 1.0000x reference)
import jax
import jax.numpy as jnp
from jax.experimental import pallas as pl


def kernel(x, weights, indices, W0, g0, b0, m0, v0, Wdw, Wpw, g1, b1, m1, v1, Wg1, gg1, bg1, mg1, vg1, Wg2, gg2, bg2, mg2, vg2):
    raise NotImplementedError("write your pallas kernel here")


# speedup vs baseline: 3.7247x; 3.7247x over previous
import functools

import jax
import jax.numpy as jnp
from jax.experimental import pallas as pl
from jax.experimental.pallas import tpu as pltpu

_B, _C, _H, _W = 2, 96, 224, 224
_CS = _C // 2
_R = 16
_NB = _H // _R
_RH = _R + 4
_KP = 9 * 128
_NC = 176
_CW = 228


def _body(sel_ref, x_ref, wcat_ref, dw_ref, pw_ref, g2_ref, bias_ref, wc_ref,
          o_ref, sc_ref):
    b = pl.program_id(0)
    i = pl.program_id(1)
    s0 = sel_ref[b, 0]
    s1 = sel_ref[b, 1]
    s2 = sel_ref[b, 2]
    wc = wc_ref[0]

    @pl.when((b == 0) & (i == 0))
    def _init():
        sc_ref[...] = jnp.zeros((_RH, _CW, _KP), jnp.bfloat16)

    xc = x_ref[0, 3:3 + _R, 4:228, :].astype(jnp.float32)
    o_ref[0] = xc * wc[3:4, :].reshape(1, 1, _C)

    @pl.when(s0 + s2 > 0)
    def _shared_matmul():
        for t in range(9):
            di, dj = t // 3, t % 3
            sc_ref[:, :, 128 * t:128 * t + _C] = (
                x_ref[0, di:di + _RH, 1 + dj:1 + dj + _CW, :])
        mm = jnp.dot(sc_ref[...].reshape(_RH * _CW, _KP), wcat_ref[...],
                     preferred_element_type=jnp.float32)
        mm3 = mm.reshape(_RH, _CW, _NC)

        @pl.when(s0 > 0)
        def _e0():
            e0l = mm3[2:2 + _R, 2:226, 0:_C] + bias_ref[0:1, :].reshape(1, 1, _C)
            o_ref[0] += (e0l * jax.nn.sigmoid(e0l)) * wc[0:1, :].reshape(1, 1, _C)

        @pl.when(s2 > 0)
        def _e2():
            yl = mm3[:, :, 128:176] + bias_ref[2:3, 0:_CS].reshape(1, 1, _CS)
            ya = yl * jax.nn.sigmoid(yl)
            y2l = jnp.zeros((_R, 224, _CS), jnp.float32)
            y2l += bias_ref[3:4, 0:_CS].reshape(1, 1, _CS)
            for t in range(25):
                u, v = t // 5, t % 5
                y2l += ya[u:u + _R, v:v + 224, :] * g2_ref[t:t + 1, :].reshape(1, 1, _CS)
            y2a = y2l * jax.nn.sigmoid(y2l)
            e2 = jnp.concatenate([ya[2:2 + _R, 2:226, :], y2a], axis=-1)
            o_ref[0] += e2 * wc[2:3, :].reshape(1, 1, _C)

    @pl.when(s1 > 0)
    def _e1():
        xir = x_ref[0, 1:1 + _RH, 2:230, _CS:_C].astype(jnp.float32)
        z = jnp.zeros((_R, 224, _CS), jnp.float32)
        for t in range(25):
            u, v = t // 5, t % 5
            z += xir[u:u + _R, v:v + 224, :] * dw_ref[t:t + 1, :].reshape(1, 1, _CS)
        p = jnp.dot(z.astype(jnp.bfloat16).reshape(_R * 224, _CS), pw_ref[...],
                    preferred_element_type=jnp.float32)
        e1l = p.reshape(_R, 224, _C) + bias_ref[1:2, :].reshape(1, 1, _C)
        o_ref[0] += (e1l * jax.nn.sigmoid(e1l)) * wc[1:2, :].reshape(1, 1, _C)


@functools.partial(jax.jit, static_argnums=())
def kernel(x, weights, indices, W0, g0, b0, m0, v0, Wdw, Wpw, g1, b1, m1, v1,
           Wg1, gg1, bg1, mg1, vg1, Wg2, gg2, bg2, mg2, vg2):
    f32 = jnp.float32
    eps = 1e-5

    s0 = g0 / jnp.sqrt(v0 + eps); t0 = b0 - m0 * s0
    W0f = W0 * s0[:, None, None, None]
    s1 = g1 / jnp.sqrt(v1 + eps); t1 = b1 - m1 * s1
    Wpwf = Wpw * s1[:, None, None, None]
    sg1 = gg1 / jnp.sqrt(vg1 + eps); tg1 = bg1 - mg1 * sg1
    Wg1f = Wg1 * sg1[:, None, None, None]
    sg2 = gg2 / jnp.sqrt(vg2 + eps); tg2 = bg2 - mg2 * sg2
    Wg2f = Wg2 * sg2[:, None, None, None]

    A0 = jnp.pad(W0f.transpose(2, 3, 1, 0), ((0, 0), (0, 0), (0, 128 - _CS), (0, 0)))
    A0 = A0.reshape(_KP, _C)
    A1 = jnp.pad(Wg1f.transpose(2, 3, 1, 0), ((0, 0), (0, 0), (0, 128 - _C), (0, 0)))
    A1 = A1.reshape(_KP, _CS)
    wcat = jnp.concatenate(
        [A0, jnp.zeros((_KP, 32), f32), A1], axis=1).astype(jnp.bfloat16)

    dwW = Wdw[:, 0].transpose(1, 2, 0).reshape(25, _CS)
    pwW = Wpwf[:, :, 0, 0].T.astype(jnp.bfloat16)
    g2w = Wg2f[:, 0].transpose(1, 2, 0).reshape(25, _CS)
    biasr = jnp.stack([t0, t1, jnp.pad(tg1, (0, _CS)), jnp.pad(tg2, (0, _CS))])

    onehot = (indices[:, :, None] == jnp.arange(4)[None, None, :]).astype(f32)
    wsum = jnp.einsum('bke,bkc->bec', onehot, weights)
    idc = 0.1 * jnp.sum(onehot[:, :, 3], axis=1)
    wcomb = wsum.at[:, 3, :].set(idc[:, None])
    selflags = (jnp.sum(onehot, axis=1) > 0).astype(jnp.int32)

    xt = jnp.pad(jnp.transpose(x, (0, 2, 3, 1)),
                 ((0, 0), (3, 3), (4, 4), (0, 0))).astype(jnp.bfloat16)

    spec = pltpu.PrefetchScalarGridSpec(
        num_scalar_prefetch=1,
        grid=(_B, _NB),
        in_specs=[
            pl.BlockSpec((pl.Element(1), pl.Element(_R + 6), pl.Element(232),
                          pl.Element(_C)), lambda b, i, *_: (b, i * _R, 0, 0)),
            pl.BlockSpec((_KP, _NC), lambda b, i, *_: (0, 0)),
            pl.BlockSpec((25, _CS), lambda b, i, *_: (0, 0)),
            pl.BlockSpec((_CS, _C), lambda b, i, *_: (0, 0)),
            pl.BlockSpec((25, _CS), lambda b, i, *_: (0, 0)),
            pl.BlockSpec((4, _C), lambda b, i, *_: (0, 0)),
            pl.BlockSpec((1, 4, _C), lambda b, i, *_: (b, 0, 0)),
        ],
        out_specs=pl.BlockSpec((1, _R, 224, _C), lambda b, i, *_: (b, i, 0, 0)),
        scratch_shapes=[pltpu.VMEM((_RH, _CW, _KP), jnp.bfloat16)],
    )
    out = pl.pallas_call(
        _body,
        grid_spec=spec,
        out_shape=jax.ShapeDtypeStruct((_B, _H, _W, _C), f32),
    )(selflags, xt, wcat, dwW, pwW, g2w, biasr, wcomb)
    return jnp.transpose(out, (0, 3, 1, 2))


# speedup vs baseline: 4.5225x; 1.2142x over previous
import functools

import jax
import jax.numpy as jnp
from jax.experimental import pallas as pl
from jax.experimental.pallas import tpu as pltpu

_B, _C, _H, _W = 2, 96, 224, 224
_CS = _C // 2
_R = 16
_NB = _H // _R
_RH = _R + 4
_KP = 9 * 128
_NC = 176
_CW = 228


def _body(sel_ref, x_ref, wcat_ref, dw_ref, pw_ref, g2_ref, bias_ref, wc_ref,
          wct_ref, o_ref, xs_ref, sc_ref, ya_ref, xir_ref, acc_ref):
    b = pl.program_id(0)
    i = pl.program_id(1)
    s0 = sel_ref[b, 0]
    s1 = sel_ref[b, 1]
    s2 = sel_ref[b, 2]
    wc = wc_ref[0]

    @pl.when((b == 0) & (i == 0))
    def _init():
        xs_ref[...] = jnp.zeros((_R + 6, 232, 128), jnp.bfloat16)
        sc_ref[...] = jnp.zeros((_RH, _CW, _KP), jnp.bfloat16)

    @pl.when(s0 + s1 + s2 > 0)
    def _experts():
        xs_ref[:, :, 0:_C] = jnp.transpose(x_ref[0, :, 0:_R + 6, :], (1, 2, 0))

        xc = xs_ref[3:3 + _R, 4:228, 0:_C].astype(jnp.float32)
        acc_ref[...] = xc * wc[3:4, :].reshape(1, 1, _C)

        @pl.when(s0 + s2 > 0)
        def _e02():
            for t in range(9):
                di, dj = t // 3, t % 3
                sc_ref[:, :, 128 * t:128 * (t + 1)] = (
                    xs_ref[di:di + _RH, 1 + dj:1 + dj + _CW, :])
            mm = jnp.dot(sc_ref[...].reshape(_RH * _CW, _KP), wcat_ref[...],
                         preferred_element_type=jnp.float32)
            mm3 = mm.reshape(_RH, _CW, _NC)

            @pl.when(s0 > 0)
            def _e0():
                e0l = mm3[2:2 + _R, 2:226, 0:_C] + bias_ref[0:1, :].reshape(1, 1, _C)
                acc_ref[...] += (e0l * jax.nn.sigmoid(e0l)) * wc[0:1, :].reshape(1, 1, _C)

            @pl.when(s2 > 0)
            def _e2():
                yl = mm3[:, :, 128:176] + bias_ref[2:3, 0:_CS].reshape(1, 1, _CS)
                ya_ref[...] = yl * jax.nn.sigmoid(yl)
                y2l = jnp.broadcast_to(
                    bias_ref[3:4, 0:_CS].reshape(1, 1, _CS), (_R, 224, _CS)
                ).astype(jnp.float32)
                for t in range(25):
                    u, v = t // 5, t % 5
                    y2l += (ya_ref[u:u + _R, v:v + 224, :]
                            * g2_ref[t:t + 1, :].reshape(1, 1, _CS))
                y2a = y2l * jax.nn.sigmoid(y2l)
                e2 = jnp.concatenate([ya_ref[2:2 + _R, 2:226, :], y2a], axis=-1)
                acc_ref[...] += e2 * wc[2:3, :].reshape(1, 1, _C)

        @pl.when(s1 > 0)
        def _e1():
            xir_ref[...] = xs_ref[1:1 + _RH, 2:230, _CS:_C]
            z = jnp.zeros((_R, 224, _CS), jnp.float32)
            for t in range(25):
                u, v = t // 5, t % 5
                z += (xir_ref[u:u + _R, v:v + 224, :].astype(jnp.float32)
                      * dw_ref[t:t + 1, :].reshape(1, 1, _CS))
            p = jnp.dot(z.astype(jnp.bfloat16).reshape(_R * 224, _CS), pw_ref[...],
                        preferred_element_type=jnp.float32)
            e1l = p.reshape(_R, 224, _C) + bias_ref[1:2, :].reshape(1, 1, _C)
            acc_ref[...] += (e1l * jax.nn.sigmoid(e1l)) * wc[1:2, :].reshape(1, 1, _C)

        o_ref[0] = jnp.transpose(acc_ref[...], (2, 0, 1))

    @pl.when(s0 + s1 + s2 == 0)
    def _identity_only():
        w3o = wct_ref[0, :, 3:4].reshape(_C, 1, 1)
        o_ref[0] = x_ref[0, :, 3:3 + _R, 4:228].astype(jnp.float32) * w3o


@functools.partial(jax.jit, static_argnums=())
def kernel(x, weights, indices, W0, g0, b0, m0, v0, Wdw, Wpw, g1, b1, m1, v1,
           Wg1, gg1, bg1, mg1, vg1, Wg2, gg2, bg2, mg2, vg2):
    f32 = jnp.float32
    eps = 1e-5

    s0 = g0 / jnp.sqrt(v0 + eps); t0 = b0 - m0 * s0
    W0f = W0 * s0[:, None, None, None]
    s1 = g1 / jnp.sqrt(v1 + eps); t1 = b1 - m1 * s1
    Wpwf = Wpw * s1[:, None, None, None]
    sg1 = gg1 / jnp.sqrt(vg1 + eps); tg1 = bg1 - mg1 * sg1
    Wg1f = Wg1 * sg1[:, None, None, None]
    sg2 = gg2 / jnp.sqrt(vg2 + eps); tg2 = bg2 - mg2 * sg2
    Wg2f = Wg2 * sg2[:, None, None, None]

    A0 = jnp.pad(W0f.transpose(2, 3, 1, 0), ((0, 0), (0, 0), (0, 128 - _CS), (0, 0)))
    A0 = A0.reshape(_KP, _C)
    A1 = jnp.pad(Wg1f.transpose(2, 3, 1, 0), ((0, 0), (0, 0), (0, 128 - _C), (0, 0)))
    A1 = A1.reshape(_KP, _CS)
    wcat = jnp.concatenate(
        [A0, jnp.zeros((_KP, 32), f32), A1], axis=1).astype(jnp.bfloat16)

    dwW = Wdw[:, 0].transpose(1, 2, 0).reshape(25, _CS)
    pwW = Wpwf[:, :, 0, 0].T.astype(jnp.bfloat16)
    g2w = Wg2f[:, 0].transpose(1, 2, 0).reshape(25, _CS)
    biasr = jnp.stack([t0, t1, jnp.pad(tg1, (0, _CS)), jnp.pad(tg2, (0, _CS))])

    onehot = (indices[:, :, None] == jnp.arange(4)[None, None, :]).astype(f32)
    wsum = jnp.einsum('bke,bkc->bec', onehot, weights)
    idc = 0.1 * jnp.sum(onehot[:, :, 3], axis=1)
    wcomb = wsum.at[:, 3, :].set(idc[:, None])
    wcombT = jnp.transpose(wcomb, (0, 2, 1))
    selflags = (jnp.sum(onehot, axis=1) > 0).astype(jnp.int32)

    xp = jnp.pad(x, ((0, 0), (0, 0), (3, 5), (4, 4))).astype(jnp.bfloat16)

    spec = pltpu.PrefetchScalarGridSpec(
        num_scalar_prefetch=1,
        grid=(_B, _NB),
        in_specs=[
            pl.BlockSpec((pl.Element(1), pl.Element(_C), pl.Element(_R + 8),
                          pl.Element(232)), lambda b, i, *_: (b, 0, i * _R, 0)),
            pl.BlockSpec((_KP, _NC), lambda b, i, *_: (0, 0)),
            pl.BlockSpec((25, _CS), lambda b, i, *_: (0, 0)),
            pl.BlockSpec((_CS, _C), lambda b, i, *_: (0, 0)),
            pl.BlockSpec((25, _CS), lambda b, i, *_: (0, 0)),
            pl.BlockSpec((4, _C), lambda b, i, *_: (0, 0)),
            pl.BlockSpec((1, 4, _C), lambda b, i, *_: (b, 0, 0)),
            pl.BlockSpec((1, _C, 4), lambda b, i, *_: (b, 0, 0)),
        ],
        out_specs=pl.BlockSpec((1, _C, _R, 224), lambda b, i, *_: (b, 0, i, 0)),
        scratch_shapes=[
            pltpu.VMEM((_R + 6, 232, 128), jnp.bfloat16),
            pltpu.VMEM((_RH, _CW, _KP), jnp.bfloat16),
            pltpu.VMEM((_RH, _CW, _CS), jnp.float32),
            pltpu.VMEM((_RH, _CW, _CS), jnp.bfloat16),
            pltpu.VMEM((_R, 224, _C), jnp.float32),
        ],
    )
    out = pl.pallas_call(
        _body,
        grid_spec=spec,
        out_shape=jax.ShapeDtypeStruct((_B, _C, _H, _W), f32),
    )(selflags, xp, wcat, dwW, pwW, g2w, biasr, wcomb, wcombT)
    return out


# speedup vs baseline: 6.3278x; 1.3992x over previous
import functools

import jax
import jax.numpy as jnp
from jax.experimental import pallas as pl
from jax.experimental.pallas import tpu as pltpu

_B, _C, _H, _W = 2, 96, 224, 224
_CS = _C // 2
_R = 16
_NB = _H // _R
_RH = _R + 4
_KP = 9 * 128
_NC = 176
_CW = 228


def _body(sel_ref, x_ref, eye_ref, wcat_ref, dw_ref, pw_ref, g2_ref, bias_ref,
          wc_ref, wct_ref, o_ref, xs_ref, sc_ref, ya_ref, xir_ref, acc_ref):
    b = pl.program_id(0)
    i = pl.program_id(1)
    s0 = sel_ref[b, 0]
    s1 = sel_ref[b, 1]
    s2 = sel_ref[b, 2]
    wc = wc_ref[0]

    @pl.when((b == 0) & (i == 0))
    def _init():
        xs_ref[...] = jnp.zeros((_R + 8, 256, 128), jnp.float32)
        sc_ref[...] = jnp.zeros((_RH, _CW, _KP), jnp.bfloat16)

    @pl.when(s0 + s1 + s2 > 0)
    def _experts():
        x2 = x_ref[0].reshape(_C, (_R + 8) * 256)
        xt = jax.lax.dot_general(x2, eye_ref[...], (((0,), (0,)), ((), ())),
                                 preferred_element_type=jnp.float32)
        xs_ref[:, :, 0:_C] = xt.reshape(_R + 8, 256, _C)

        xc = xs_ref[3:3 + _R, 16:240, 0:_C]
        acc_ref[...] = xc * wc[3:4, :].reshape(1, 1, _C)

        @pl.when(s0 + s2 > 0)
        def _e02():
            for t in range(9):
                di, dj = t // 3, t % 3
                sc_ref[:, :, 128 * t:128 * (t + 1)] = (
                    xs_ref[di:di + _RH, 13 + dj:13 + dj + _CW, :].astype(jnp.bfloat16))
            mm = jnp.dot(sc_ref[...].reshape(_RH * _CW, _KP), wcat_ref[...],
                         preferred_element_type=jnp.float32)
            mm3 = mm.reshape(_RH, _CW, _NC)

            @pl.when(s0 > 0)
            def _e0():
                e0l = mm3[2:2 + _R, 2:226, 0:_C] + bias_ref[0:1, :].reshape(1, 1, _C)
                acc_ref[...] += (e0l * jax.nn.sigmoid(e0l)) * wc[0:1, :].reshape(1, 1, _C)

            @pl.when(s2 > 0)
            def _e2():
                yl = mm3[:, :, 128:176] + bias_ref[2:3, 0:_CS].reshape(1, 1, _CS)
                ya_ref[...] = yl * jax.nn.sigmoid(yl)
                y2l = jnp.broadcast_to(
                    bias_ref[3:4, 0:_CS].reshape(1, 1, _CS), (_R, 224, _CS)
                ).astype(jnp.float32)
                for t in range(25):
                    u, v = t // 5, t % 5
                    y2l += (ya_ref[u:u + _R, v:v + 224, :]
                            * g2_ref[t:t + 1, :].reshape(1, 1, _CS))
                y2a = y2l * jax.nn.sigmoid(y2l)
                e2 = jnp.concatenate([ya_ref[2:2 + _R, 2:226, :], y2a], axis=-1)
                acc_ref[...] += e2 * wc[2:3, :].reshape(1, 1, _C)

        @pl.when(s1 > 0)
        def _e1():
            xir_ref[...] = xs_ref[1:1 + _RH, 14:242, _CS:_C].astype(jnp.bfloat16)
            z = jnp.zeros((_R, 224, _CS), jnp.float32)
            for t in range(25):
                u, v = t // 5, t % 5
                z += (xir_ref[u:u + _R, v:v + 224, :].astype(jnp.float32)
                      * dw_ref[t:t + 1, :].reshape(1, 1, _CS))
            p = jnp.dot(z.astype(jnp.bfloat16).reshape(_R * 224, _CS), pw_ref[...],
                        preferred_element_type=jnp.float32)
            e1l = p.reshape(_R, 224, _C) + bias_ref[1:2, :].reshape(1, 1, _C)
            acc_ref[...] += (e1l * jax.nn.sigmoid(e1l)) * wc[1:2, :].reshape(1, 1, _C)

        o_ref[0] = jnp.transpose(acc_ref[...], (2, 0, 1))

    @pl.when(s0 + s1 + s2 == 0)
    def _identity_only():
        w3o = wct_ref[0, :, 3:4].reshape(_C, 1, 1)
        o_ref[0] = x_ref[0, :, 3:3 + _R, 16:240].astype(jnp.float32) * w3o


@functools.partial(jax.jit, static_argnums=())
def kernel(x, weights, indices, W0, g0, b0, m0, v0, Wdw, Wpw, g1, b1, m1, v1,
           Wg1, gg1, bg1, mg1, vg1, Wg2, gg2, bg2, mg2, vg2):
    f32 = jnp.float32
    eps = 1e-5

    s0 = g0 / jnp.sqrt(v0 + eps); t0 = b0 - m0 * s0
    W0f = W0 * s0[:, None, None, None]
    s1 = g1 / jnp.sqrt(v1 + eps); t1 = b1 - m1 * s1
    Wpwf = Wpw * s1[:, None, None, None]
    sg1 = gg1 / jnp.sqrt(vg1 + eps); tg1 = bg1 - mg1 * sg1
    Wg1f = Wg1 * sg1[:, None, None, None]
    sg2 = gg2 / jnp.sqrt(vg2 + eps); tg2 = bg2 - mg2 * sg2
    Wg2f = Wg2 * sg2[:, None, None, None]

    A0 = jnp.pad(W0f.transpose(2, 3, 1, 0), ((0, 0), (0, 0), (0, 128 - _CS), (0, 0)))
    A0 = A0.reshape(_KP, _C)
    A1 = jnp.pad(Wg1f.transpose(2, 3, 1, 0), ((0, 0), (0, 0), (0, 128 - _C), (0, 0)))
    A1 = A1.reshape(_KP, _CS)
    wcat = jnp.concatenate(
        [A0, jnp.zeros((_KP, 32), f32), A1], axis=1).astype(jnp.bfloat16)

    dwW = Wdw[:, 0].transpose(1, 2, 0).reshape(25, _CS)
    pwW = Wpwf[:, :, 0, 0].T.astype(jnp.bfloat16)
    g2w = Wg2f[:, 0].transpose(1, 2, 0).reshape(25, _CS)
    biasr = jnp.stack([t0, t1, jnp.pad(tg1, (0, _CS)), jnp.pad(tg2, (0, _CS))])

    onehot = (indices[:, :, None] == jnp.arange(4)[None, None, :]).astype(f32)
    wsum = jnp.einsum('bke,bkc->bec', onehot, weights)
    idc = 0.1 * jnp.sum(onehot[:, :, 3], axis=1)
    wcomb = wsum.at[:, 3, :].set(idc[:, None])
    wcombT = jnp.transpose(wcomb, (0, 2, 1))
    selflags = (jnp.sum(onehot, axis=1) > 0).astype(jnp.int32)

    xp = jnp.pad(x, ((0, 0), (0, 0), (3, 5), (16, 16))).astype(jnp.bfloat16)
    eye = jnp.eye(_C, dtype=jnp.bfloat16)

    spec = pltpu.PrefetchScalarGridSpec(
        num_scalar_prefetch=1,
        grid=(_B, _NB),
        in_specs=[
            pl.BlockSpec((pl.Element(1), pl.Element(_C), pl.Element(_R + 8),
                          pl.Element(256)), lambda b, i, *_: (b, 0, i * _R, 0)),
            pl.BlockSpec((_C, _C), lambda b, i, *_: (0, 0)),
            pl.BlockSpec((_KP, _NC), lambda b, i, *_: (0, 0)),
            pl.BlockSpec((25, _CS), lambda b, i, *_: (0, 0)),
            pl.BlockSpec((_CS, _C), lambda b, i, *_: (0, 0)),
            pl.BlockSpec((25, _CS), lambda b, i, *_: (0, 0)),
            pl.BlockSpec((4, _C), lambda b, i, *_: (0, 0)),
            pl.BlockSpec((1, 4, _C), lambda b, i, *_: (b, 0, 0)),
            pl.BlockSpec((1, _C, 4), lambda b, i, *_: (b, 0, 0)),
        ],
        out_specs=pl.BlockSpec((1, _C, _R, 224), lambda b, i, *_: (b, 0, i, 0)),
        scratch_shapes=[
            pltpu.VMEM((_R + 8, 256, 128), jnp.float32),
            pltpu.VMEM((_RH, _CW, _KP), jnp.bfloat16),
            pltpu.VMEM((_RH, _CW, _CS), jnp.float32),
            pltpu.VMEM((_RH, _CW, _CS), jnp.bfloat16),
            pltpu.VMEM((_R, 224, _C), jnp.float32),
        ],
    )
    out = pl.pallas_call(
        _body,
        grid_spec=spec,
        out_shape=jax.ShapeDtypeStruct((_B, _C, _H, _W), f32),
    )(selflags, xp, eye, wcat, dwW, pwW, g2w, biasr, wcomb, wcombT)
    return out


# speedup vs baseline: 6.9694x; 1.1014x over previous
import functools

import jax
import jax.numpy as jnp
from jax.experimental import pallas as pl
from jax.experimental.pallas import tpu as pltpu

_B, _C, _H, _W = 2, 96, 224, 224
_CS = _C // 2
_R = 16
_NB = _H // _R
_RH = _R + 4
_KP = 9 * 128
_NC = 176
_CW = 228


def _pad_body(x_ref, o_ref):
    o_ref[...] = jnp.zeros(o_ref.shape, jnp.bfloat16)
    o_ref[0, :, 3:227, 16:240] = x_ref[0].astype(jnp.bfloat16)


def _pad_cast(x):
    return pl.pallas_call(
        _pad_body,
        grid=(_B, 6),
        in_specs=[pl.BlockSpec((1, 16, 224, 224), lambda b, c: (b, c, 0, 0))],
        out_specs=pl.BlockSpec((1, 16, 232, 256), lambda b, c: (b, c, 0, 0)),
        out_shape=jax.ShapeDtypeStruct((_B, _C, 232, 256), jnp.bfloat16),
    )(x)


def _body(sel_ref, x_ref, eye_ref, wcat_ref, dw_ref, pw_ref, g2_ref, bias_ref,
          wc_ref, wct_ref, o_ref, xs_ref, sc_ref, ya_ref, xir_ref, acc_ref):
    b = pl.program_id(0)
    i = pl.program_id(1)
    s0 = sel_ref[b, 0]
    s1 = sel_ref[b, 1]
    s2 = sel_ref[b, 2]
    wc = wc_ref[0]

    @pl.when((b == 0) & (i == 0))
    def _init():
        xs_ref[...] = jnp.zeros((_R + 8, 256, 128), jnp.float32)
        sc_ref[...] = jnp.zeros((_RH, _CW, _KP), jnp.bfloat16)

    @pl.when(s0 + s1 + s2 > 0)
    def _experts():
        x2 = x_ref[0].reshape(_C, (_R + 8) * 256)
        xt = jax.lax.dot_general(x2, eye_ref[...], (((0,), (0,)), ((), ())),
                                 preferred_element_type=jnp.float32)
        xs_ref[:, :, 0:_C] = xt.reshape(_R + 8, 256, _C)

        xc = xs_ref[3:3 + _R, 16:240, 0:_C]
        acc_ref[...] = xc * wc[3:4, :].reshape(1, 1, _C)

        @pl.when(s0 + s2 > 0)
        def _e02():
            for t in range(9):
                di, dj = t // 3, t % 3
                sc_ref[:, :, 128 * t:128 * (t + 1)] = (
                    xs_ref[di:di + _RH, 13 + dj:13 + dj + _CW, :].astype(jnp.bfloat16))
            mm = jnp.dot(sc_ref[...].reshape(_RH * _CW, _KP), wcat_ref[...],
                         preferred_element_type=jnp.float32)
            mm3 = mm.reshape(_RH, _CW, _NC)

            @pl.when(s0 > 0)
            def _e0():
                e0l = mm3[2:2 + _R, 2:226, 0:_C] + bias_ref[0:1, :].reshape(1, 1, _C)
                acc_ref[...] += (e0l * jax.nn.sigmoid(e0l)) * wc[0:1, :].reshape(1, 1, _C)

            @pl.when(s2 > 0)
            def _e2():
                yl = mm3[:, :, 128:176] + bias_ref[2:3, 0:_CS].reshape(1, 1, _CS)
                ya_ref[...] = yl * jax.nn.sigmoid(yl)
                y2l = jnp.broadcast_to(
                    bias_ref[3:4, 0:_CS].reshape(1, 1, _CS), (_R, 224, _CS)
                ).astype(jnp.float32)
                for t in range(25):
                    u, v = t // 5, t % 5
                    y2l += (ya_ref[u:u + _R, v:v + 224, :]
                            * g2_ref[t:t + 1, :].reshape(1, 1, _CS))
                y2a = y2l * jax.nn.sigmoid(y2l)
                e2 = jnp.concatenate([ya_ref[2:2 + _R, 2:226, :], y2a], axis=-1)
                acc_ref[...] += e2 * wc[2:3, :].reshape(1, 1, _C)

        @pl.when(s1 > 0)
        def _e1():
            xir_ref[...] = xs_ref[1:1 + _RH, 14:242, _CS:_C].astype(jnp.bfloat16)
            z = jnp.zeros((_R, 224, _CS), jnp.float32)
            for t in range(25):
                u, v = t // 5, t % 5
                z += (xir_ref[u:u + _R, v:v + 224, :].astype(jnp.float32)
                      * dw_ref[t:t + 1, :].reshape(1, 1, _CS))
            p = jnp.dot(z.astype(jnp.bfloat16).reshape(_R * 224, _CS), pw_ref[...],
                        preferred_element_type=jnp.float32)
            e1l = p.reshape(_R, 224, _C) + bias_ref[1:2, :].reshape(1, 1, _C)
            acc_ref[...] += (e1l * jax.nn.sigmoid(e1l)) * wc[1:2, :].reshape(1, 1, _C)

        o_ref[0] = jnp.transpose(acc_ref[...], (2, 0, 1))

    @pl.when(s0 + s1 + s2 == 0)
    def _identity_only():
        w3o = wct_ref[0, :, 3:4].reshape(_C, 1, 1)
        o_ref[0] = x_ref[0, :, 3:3 + _R, 16:240].astype(jnp.float32) * w3o


@functools.partial(jax.jit, static_argnums=())
def kernel(x, weights, indices, W0, g0, b0, m0, v0, Wdw, Wpw, g1, b1, m1, v1,
           Wg1, gg1, bg1, mg1, vg1, Wg2, gg2, bg2, mg2, vg2):
    f32 = jnp.float32
    eps = 1e-5

    s0 = g0 / jnp.sqrt(v0 + eps); t0 = b0 - m0 * s0
    W0f = W0 * s0[:, None, None, None]
    s1 = g1 / jnp.sqrt(v1 + eps); t1 = b1 - m1 * s1
    Wpwf = Wpw * s1[:, None, None, None]
    sg1 = gg1 / jnp.sqrt(vg1 + eps); tg1 = bg1 - mg1 * sg1
    Wg1f = Wg1 * sg1[:, None, None, None]
    sg2 = gg2 / jnp.sqrt(vg2 + eps); tg2 = bg2 - mg2 * sg2
    Wg2f = Wg2 * sg2[:, None, None, None]

    A0 = jnp.pad(W0f.transpose(2, 3, 1, 0), ((0, 0), (0, 0), (0, 128 - _CS), (0, 0)))
    A0 = A0.reshape(_KP, _C)
    A1 = jnp.pad(Wg1f.transpose(2, 3, 1, 0), ((0, 0), (0, 0), (0, 128 - _C), (0, 0)))
    A1 = A1.reshape(_KP, _CS)
    wcat = jnp.concatenate(
        [A0, jnp.zeros((_KP, 32), f32), A1], axis=1).astype(jnp.bfloat16)

    dwW = Wdw[:, 0].transpose(1, 2, 0).reshape(25, _CS)
    pwW = Wpwf[:, :, 0, 0].T.astype(jnp.bfloat16)
    g2w = Wg2f[:, 0].transpose(1, 2, 0).reshape(25, _CS)
    biasr = jnp.stack([t0, t1, jnp.pad(tg1, (0, _CS)), jnp.pad(tg2, (0, _CS))])

    onehot = (indices[:, :, None] == jnp.arange(4)[None, None, :]).astype(f32)
    wsum = jnp.einsum('bke,bkc->bec', onehot, weights)
    idc = 0.1 * jnp.sum(onehot[:, :, 3], axis=1)
    wcomb = wsum.at[:, 3, :].set(idc[:, None])
    wcombT = jnp.transpose(wcomb, (0, 2, 1))
    selflags = (jnp.sum(onehot, axis=1) > 0).astype(jnp.int32)

    xp = _pad_cast(x)
    eye = jnp.eye(_C, dtype=jnp.bfloat16)

    spec = pltpu.PrefetchScalarGridSpec(
        num_scalar_prefetch=1,
        grid=(_B, _NB),
        in_specs=[
            pl.BlockSpec((pl.Element(1), pl.Element(_C), pl.Element(_R + 8),
                          pl.Element(256)), lambda b, i, *_: (b, 0, i * _R, 0)),
            pl.BlockSpec((_C, _C), lambda b, i, *_: (0, 0)),
            pl.BlockSpec((_KP, _NC), lambda b, i, *_: (0, 0)),
            pl.BlockSpec((25, _CS), lambda b, i, *_: (0, 0)),
            pl.BlockSpec((_CS, _C), lambda b, i, *_: (0, 0)),
            pl.BlockSpec((25, _CS), lambda b, i, *_: (0, 0)),
            pl.BlockSpec((4, _C), lambda b, i, *_: (0, 0)),
            pl.BlockSpec((1, 4, _C), lambda b, i, *_: (b, 0, 0)),
            pl.BlockSpec((1, _C, 4), lambda b, i, *_: (b, 0, 0)),
        ],
        out_specs=pl.BlockSpec((1, _C, _R, 224), lambda b, i, *_: (b, 0, i, 0)),
        scratch_shapes=[
            pltpu.VMEM((_R + 8, 256, 128), jnp.float32),
            pltpu.VMEM((_RH, _CW, _KP), jnp.bfloat16),
            pltpu.VMEM((_RH, _CW, _CS), jnp.float32),
            pltpu.VMEM((_RH, _CW, _CS), jnp.bfloat16),
            pltpu.VMEM((_R, 224, _C), jnp.float32),
        ],
    )
    out = pl.pallas_call(
        _body,
        grid_spec=spec,
        out_shape=jax.ShapeDtypeStruct((_B, _C, _H, _W), f32),
    )(selflags, xp, eye, wcat, dwW, pwW, g2w, biasr, wcomb, wcombT)
    return out


# speedup vs baseline: 7.1185x; 1.0214x over previous
import functools

import jax
import jax.numpy as jnp
from jax.experimental import pallas as pl
from jax.experimental.pallas import tpu as pltpu

_B, _C, _H, _W = 2, 96, 224, 224
_CS = _C // 2
_R = 16
_NB = _H // _R
_RH = _R + 4
_KP = 9 * 128
_NC = 176
_CW = 228


def _pad_body(x_ref, o_ref):
    o_ref[...] = jnp.zeros(o_ref.shape, jnp.bfloat16)
    o_ref[0, :, 3:227, 16:240] = x_ref[0].astype(jnp.bfloat16)


def _pad_cast(x):
    return pl.pallas_call(
        _pad_body,
        grid=(_B, 6),
        in_specs=[pl.BlockSpec((1, 16, 224, 224), lambda b, c: (b, c, 0, 0))],
        out_specs=pl.BlockSpec((1, 16, 232, 256), lambda b, c: (b, c, 0, 0)),
        out_shape=jax.ShapeDtypeStruct((_B, _C, 232, 256), jnp.bfloat16),
    )(x)


def _body(sel_ref, x_ref, eye_ref, wc0_ref, wc1_ref, dw_ref, pw_ref, g2_ref,
          bias_ref, wc_ref, wct_ref, o_ref, xs_ref, sc_ref, ya_ref, xir_ref,
          acc_ref):
    b = pl.program_id(0)
    i = pl.program_id(1)
    s0 = sel_ref[b, 0]
    s1 = sel_ref[b, 1]
    s2 = sel_ref[b, 2]
    wc = wc_ref[0]

    @pl.when((b == 0) & (i == 0))
    def _init():
        xs_ref[...] = jnp.zeros((_R + 8, 256, 128), jnp.float32)
        sc_ref[...] = jnp.zeros((_RH, _CW, _KP), jnp.bfloat16)

    @pl.when(s0 + s1 + s2 > 0)
    def _experts():
        x2 = x_ref[0].reshape(_C, (_R + 8) * 256)
        xt = jax.lax.dot_general(x2, eye_ref[...], (((0,), (0,)), ((), ())),
                                 preferred_element_type=jnp.float32)
        xs_ref[:, :, 0:_C] = xt.reshape(_R + 8, 256, _C)

        xc = xs_ref[3:3 + _R, 16:240, 0:_C]
        acc_ref[...] = xc * wc[3:4, :].reshape(1, 1, _C)

        @pl.when(s0 + s2 > 0)
        def _e02():
            for t in range(9):
                di, dj = t // 3, t % 3
                sc_ref[:, :, 128 * t:128 * (t + 1)] = (
                    xs_ref[di:di + _RH, 13 + dj:13 + dj + _CW, :].astype(jnp.bfloat16))
            @pl.when(s0 > 0)
            def _e0():
                mme = jnp.dot(sc_ref[...].reshape(_RH * _CW, _KP), wc0_ref[...],
                              preferred_element_type=jnp.float32
                              ).reshape(_RH, _CW, _C)
                e0l = mme[2:2 + _R, 2:226, :] + bias_ref[0:1, :].reshape(1, 1, _C)
                acc_ref[...] += (e0l * jax.nn.sigmoid(e0l)) * wc[0:1, :].reshape(1, 1, _C)

            @pl.when(s2 > 0)
            def _e2():
                mmc = jnp.dot(sc_ref[...].reshape(_RH * _CW, _KP), wc1_ref[...],
                              preferred_element_type=jnp.float32
                              ).reshape(_RH, _CW, _CS)
                yl = mmc + bias_ref[2:3, 0:_CS].reshape(1, 1, _CS)
                ya_ref[...] = yl * jax.nn.sigmoid(yl)
                y2l = jnp.broadcast_to(
                    bias_ref[3:4, 0:_CS].reshape(1, 1, _CS), (_R, 224, _CS)
                ).astype(jnp.float32)
                for t in range(25):
                    u, v = t // 5, t % 5
                    y2l += (ya_ref[u:u + _R, v:v + 224, :]
                            * g2_ref[t:t + 1, :].reshape(1, 1, _CS))
                y2a = y2l * jax.nn.sigmoid(y2l)
                e2 = jnp.concatenate([ya_ref[2:2 + _R, 2:226, :], y2a], axis=-1)
                acc_ref[...] += e2 * wc[2:3, :].reshape(1, 1, _C)

        @pl.when(s1 > 0)
        def _e1():
            xir_ref[...] = xs_ref[1:1 + _RH, 14:242, _CS:_C].astype(jnp.bfloat16)
            z = jnp.zeros((_R, 224, _CS), jnp.float32)
            for t in range(25):
                u, v = t // 5, t % 5
                z += (xir_ref[u:u + _R, v:v + 224, :].astype(jnp.float32)
                      * dw_ref[t:t + 1, :].reshape(1, 1, _CS))
            p = jnp.dot(z.astype(jnp.bfloat16).reshape(_R * 224, _CS), pw_ref[...],
                        preferred_element_type=jnp.float32)
            e1l = p.reshape(_R, 224, _C) + bias_ref[1:2, :].reshape(1, 1, _C)
            acc_ref[...] += (e1l * jax.nn.sigmoid(e1l)) * wc[1:2, :].reshape(1, 1, _C)

        o_ref[0] = jnp.transpose(acc_ref[...], (2, 0, 1))

    @pl.when(s0 + s1 + s2 == 0)
    def _identity_only():
        w3o = wct_ref[0, :, 3:4].reshape(_C, 1, 1)
        o_ref[0] = x_ref[0, :, 3:3 + _R, 16:240].astype(jnp.float32) * w3o


@functools.partial(jax.jit, static_argnums=())
def kernel(x, weights, indices, W0, g0, b0, m0, v0, Wdw, Wpw, g1, b1, m1, v1,
           Wg1, gg1, bg1, mg1, vg1, Wg2, gg2, bg2, mg2, vg2):
    f32 = jnp.float32
    eps = 1e-5

    s0 = g0 / jnp.sqrt(v0 + eps); t0 = b0 - m0 * s0
    W0f = W0 * s0[:, None, None, None]
    s1 = g1 / jnp.sqrt(v1 + eps); t1 = b1 - m1 * s1
    Wpwf = Wpw * s1[:, None, None, None]
    sg1 = gg1 / jnp.sqrt(vg1 + eps); tg1 = bg1 - mg1 * sg1
    Wg1f = Wg1 * sg1[:, None, None, None]
    sg2 = gg2 / jnp.sqrt(vg2 + eps); tg2 = bg2 - mg2 * sg2
    Wg2f = Wg2 * sg2[:, None, None, None]

    A0 = jnp.pad(W0f.transpose(2, 3, 1, 0), ((0, 0), (0, 0), (0, 128 - _CS), (0, 0)))
    A0 = A0.reshape(_KP, _C)
    A1 = jnp.pad(Wg1f.transpose(2, 3, 1, 0), ((0, 0), (0, 0), (0, 128 - _C), (0, 0)))
    A1 = A1.reshape(_KP, _CS)
    wcat0 = A0.astype(jnp.bfloat16)
    wcat1 = A1.astype(jnp.bfloat16)

    dwW = Wdw[:, 0].transpose(1, 2, 0).reshape(25, _CS)
    pwW = Wpwf[:, :, 0, 0].T.astype(jnp.bfloat16)
    g2w = Wg2f[:, 0].transpose(1, 2, 0).reshape(25, _CS)
    biasr = jnp.stack([t0, t1, jnp.pad(tg1, (0, _CS)), jnp.pad(tg2, (0, _CS))])

    onehot = (indices[:, :, None] == jnp.arange(4)[None, None, :]).astype(f32)
    wsum = jnp.einsum('bke,bkc->bec', onehot, weights)
    idc = 0.1 * jnp.sum(onehot[:, :, 3], axis=1)
    wcomb = wsum.at[:, 3, :].set(idc[:, None])
    wcombT = jnp.transpose(wcomb, (0, 2, 1))
    selflags = (jnp.sum(onehot, axis=1) > 0).astype(jnp.int32)

    xp = _pad_cast(x)
    eye = jnp.eye(_C, dtype=jnp.bfloat16)

    spec = pltpu.PrefetchScalarGridSpec(
        num_scalar_prefetch=1,
        grid=(_B, _NB),
        in_specs=[
            pl.BlockSpec((pl.Element(1), pl.Element(_C), pl.Element(_R + 8),
                          pl.Element(256)), lambda b, i, *_: (b, 0, i * _R, 0)),
            pl.BlockSpec((_C, _C), lambda b, i, *_: (0, 0)),
            pl.BlockSpec((_KP, _C), lambda b, i, *_: (0, 0)),
            pl.BlockSpec((_KP, _CS), lambda b, i, *_: (0, 0)),
            pl.BlockSpec((25, _CS), lambda b, i, *_: (0, 0)),
            pl.BlockSpec((_CS, _C), lambda b, i, *_: (0, 0)),
            pl.BlockSpec((25, _CS), lambda b, i, *_: (0, 0)),
            pl.BlockSpec((4, _C), lambda b, i, *_: (0, 0)),
            pl.BlockSpec((1, 4, _C), lambda b, i, *_: (b, 0, 0)),
            pl.BlockSpec((1, _C, 4), lambda b, i, *_: (b, 0, 0)),
        ],
        out_specs=pl.BlockSpec((1, _C, _R, 224), lambda b, i, *_: (b, 0, i, 0)),
        scratch_shapes=[
            pltpu.VMEM((_R + 8, 256, 128), jnp.float32),
            pltpu.VMEM((_RH, _CW, _KP), jnp.bfloat16),
            pltpu.VMEM((_RH, _CW, _CS), jnp.float32),
            pltpu.VMEM((_RH, _CW, _CS), jnp.bfloat16),
            pltpu.VMEM((_R, 224, _C), jnp.float32),
        ],
    )
    out = pl.pallas_call(
        _body,
        grid_spec=spec,
        out_shape=jax.ShapeDtypeStruct((_B, _C, _H, _W), f32),
    )(selflags, xp, eye, wcat0, wcat1, dwW, pwW, g2w, biasr, wcomb, wcombT)
    return out


# speedup vs baseline: 7.8512x; 1.1029x over previous
import functools

import jax
import jax.numpy as jnp
from jax.experimental import pallas as pl
from jax.experimental.pallas import tpu as pltpu

_B, _C, _H, _W = 2, 96, 224, 224
_CS = _C // 2
_R = 16
_NB = _H // _R
_RH = _R + 4
_KP = 9 * 128
_NC = 176
_CW = 228


def _pad_body(x_ref, o_ref):
    o_ref[...] = jnp.zeros(o_ref.shape, jnp.bfloat16)
    o_ref[0, :, 3:227, 16:240] = x_ref[0].astype(jnp.bfloat16)


def _pad_cast(x):
    return pl.pallas_call(
        _pad_body,
        grid=(_B, 6),
        in_specs=[pl.BlockSpec((1, 16, 224, 224), lambda b, c: (b, c, 0, 0))],
        out_specs=pl.BlockSpec((1, 16, 232, 256), lambda b, c: (b, c, 0, 0)),
        out_shape=jax.ShapeDtypeStruct((_B, _C, 232, 256), jnp.bfloat16),
    )(x)


def _body(sel_ref, x_ref, eye_ref, wc0_ref, wc1_ref, dw_ref, pw_ref, g2_ref,
          bias_ref, wc_ref, wct_ref, o_ref, xs_ref, sc_ref, ya_ref, yp_ref,
          ysw_ref, y2u_ref, xir_ref, acc_ref):
    b = pl.program_id(0)
    i = pl.program_id(1)
    s0 = sel_ref[b, 0]
    s1 = sel_ref[b, 1]
    s2 = sel_ref[b, 2]
    wc = wc_ref[0]

    @pl.when((b == 0) & (i == 0))
    def _init():
        xs_ref[...] = jnp.zeros((_R + 8, 256, 128), jnp.float32)
        sc_ref[...] = jnp.zeros((_RH, _CW, _KP), jnp.bfloat16)

    @pl.when(s0 + s1 + s2 > 0)
    def _experts():
        x2 = x_ref[0].reshape(_C, (_R + 8) * 256)
        xt = jax.lax.dot_general(x2, eye_ref[...], (((0,), (0,)), ((), ())),
                                 preferred_element_type=jnp.float32)
        xs_ref[:, :, 0:_C] = xt.reshape(_R + 8, 256, _C)

        xc = xs_ref[3:3 + _R, 16:240, 0:_C]
        acc_ref[...] = xc * wc[3:4, :].reshape(1, 1, _C)

        @pl.when(s0 + s2 > 0)
        def _e02():
            for t in range(9):
                di, dj = t // 3, t % 3
                sc_ref[:, :, 128 * t:128 * (t + 1)] = (
                    xs_ref[di:di + _RH, 13 + dj:13 + dj + _CW, :].astype(jnp.bfloat16))
            @pl.when(s0 > 0)
            def _e0():
                mme = jnp.dot(sc_ref[...].reshape(_RH * _CW, _KP), wc0_ref[...],
                              preferred_element_type=jnp.float32
                              ).reshape(_RH, _CW, _C)
                e0l = mme[2:2 + _R, 2:226, :] + bias_ref[0:1, :].reshape(1, 1, _C)
                acc_ref[...] += (e0l * jax.nn.sigmoid(e0l)) * wc[0:1, :].reshape(1, 1, _C)

            @pl.when(s2 > 0)
            def _e2():
                mmc = jnp.dot(sc_ref[...].reshape(_RH * _CW, _KP), wc1_ref[...],
                              preferred_element_type=jnp.float32
                              ).reshape(_RH, _CW, _CS)
                yl = mmc + bias_ref[2:3, 0:_CS].reshape(1, 1, _CS)
                ya_ref[...] = yl * jax.nn.sigmoid(yl)
                yp_ref[:, :, 0:_CS] = ya_ref[:, 0:_CW:2, :]
                yp_ref[:, :, _CS:_C] = ya_ref[:, 1:_CW:2, :]
                ysw_ref[:, 0:113, 0:_CS] = yp_ref[:, 0:113, _CS:_C]
                ysw_ref[:, 0:113, _CS:_C] = yp_ref[:, 1:114, 0:_CS]
                y2p = jnp.broadcast_to(
                    bias_ref[3:4, :].reshape(1, 1, _C), (_R, 112, _C)
                ).astype(jnp.float32)
                for u in range(5):
                    y2p += yp_ref[u:u + _R, 0:112, :] * g2_ref[5 * u:5 * u + 1, :].reshape(1, 1, _C)
                    y2p += ysw_ref[u:u + _R, 0:112, :] * g2_ref[5 * u + 1:5 * u + 2, :].reshape(1, 1, _C)
                    y2p += yp_ref[u:u + _R, 1:113, :] * g2_ref[5 * u + 2:5 * u + 3, :].reshape(1, 1, _C)
                    y2p += ysw_ref[u:u + _R, 1:113, :] * g2_ref[5 * u + 3:5 * u + 4, :].reshape(1, 1, _C)
                    y2p += yp_ref[u:u + _R, 2:114, :] * g2_ref[5 * u + 4:5 * u + 5, :].reshape(1, 1, _C)
                y2a = y2p * jax.nn.sigmoid(y2p)
                y2u_ref[:, 0:224:2, :] = y2a[:, :, 0:_CS]
                y2u_ref[:, 1:224:2, :] = y2a[:, :, _CS:_C]
                e2 = jnp.concatenate([ya_ref[2:2 + _R, 2:226, :], y2u_ref[...]],
                                     axis=-1)
                acc_ref[...] += e2 * wc[2:3, :].reshape(1, 1, _C)

        @pl.when(s1 > 0)
        def _e1():
            xir_ref[...] = xs_ref[1:1 + _RH, 14:242, _CS:_C].astype(jnp.bfloat16)
            z = jnp.zeros((_R, 224, _CS), jnp.float32)
            for t in range(25):
                u, v = t // 5, t % 5
                z += (xir_ref[u:u + _R, v:v + 224, :].astype(jnp.float32)
                      * dw_ref[t:t + 1, :].reshape(1, 1, _CS))
            p = jnp.dot(z.astype(jnp.bfloat16).reshape(_R * 224, _CS), pw_ref[...],
                        preferred_element_type=jnp.float32)
            e1l = p.reshape(_R, 224, _C) + bias_ref[1:2, :].reshape(1, 1, _C)
            acc_ref[...] += (e1l * jax.nn.sigmoid(e1l)) * wc[1:2, :].reshape(1, 1, _C)

        o_ref[0] = jnp.transpose(acc_ref[...], (2, 0, 1))

    @pl.when(s0 + s1 + s2 == 0)
    def _identity_only():
        w3o = wct_ref[0, :, 3:4].reshape(_C, 1, 1)
        o_ref[0] = x_ref[0, :, 3:3 + _R, 16:240].astype(jnp.float32) * w3o


@functools.partial(jax.jit, static_argnums=())
def kernel(x, weights, indices, W0, g0, b0, m0, v0, Wdw, Wpw, g1, b1, m1, v1,
           Wg1, gg1, bg1, mg1, vg1, Wg2, gg2, bg2, mg2, vg2):
    f32 = jnp.float32
    eps = 1e-5

    s0 = g0 / jnp.sqrt(v0 + eps); t0 = b0 - m0 * s0
    W0f = W0 * s0[:, None, None, None]
    s1 = g1 / jnp.sqrt(v1 + eps); t1 = b1 - m1 * s1
    Wpwf = Wpw * s1[:, None, None, None]
    sg1 = gg1 / jnp.sqrt(vg1 + eps); tg1 = bg1 - mg1 * sg1
    Wg1f = Wg1 * sg1[:, None, None, None]
    sg2 = gg2 / jnp.sqrt(vg2 + eps); tg2 = bg2 - mg2 * sg2
    Wg2f = Wg2 * sg2[:, None, None, None]

    A0 = jnp.pad(W0f.transpose(2, 3, 1, 0), ((0, 0), (0, 0), (0, 128 - _CS), (0, 0)))
    A0 = A0.reshape(_KP, _C)
    A1 = jnp.pad(Wg1f.transpose(2, 3, 1, 0), ((0, 0), (0, 0), (0, 128 - _C), (0, 0)))
    A1 = A1.reshape(_KP, _CS)
    wcat0 = A0.astype(jnp.bfloat16)
    wcat1 = A1.astype(jnp.bfloat16)

    dwW = Wdw[:, 0].transpose(1, 2, 0).reshape(25, _CS)
    pwW = Wpwf[:, :, 0, 0].T.astype(jnp.bfloat16)
    g2w = jnp.tile(Wg2f[:, 0].transpose(1, 2, 0).reshape(25, _CS), (1, 2))
    biasr = jnp.stack([t0, t1, jnp.pad(tg1, (0, _CS)),
                       jnp.concatenate([tg2, tg2])])

    onehot = (indices[:, :, None] == jnp.arange(4)[None, None, :]).astype(f32)
    wsum = jnp.einsum('bke,bkc->bec', onehot, weights)
    idc = 0.1 * jnp.sum(onehot[:, :, 3], axis=1)
    wcomb = wsum.at[:, 3, :].set(idc[:, None])
    wcombT = jnp.transpose(wcomb, (0, 2, 1))
    selflags = (jnp.sum(onehot, axis=1) > 0).astype(jnp.int32)

    xp = _pad_cast(x)
    eye = jnp.eye(_C, dtype=jnp.bfloat16)

    spec = pltpu.PrefetchScalarGridSpec(
        num_scalar_prefetch=1,
        grid=(_B, _NB),
        in_specs=[
            pl.BlockSpec((pl.Element(1), pl.Element(_C), pl.Element(_R + 8),
                          pl.Element(256)), lambda b, i, *_: (b, 0, i * _R, 0)),
            pl.BlockSpec((_C, _C), lambda b, i, *_: (0, 0)),
            pl.BlockSpec((_KP, _C), lambda b, i, *_: (0, 0)),
            pl.BlockSpec((_KP, _CS), lambda b, i, *_: (0, 0)),
            pl.BlockSpec((25, _CS), lambda b, i, *_: (0, 0)),
            pl.BlockSpec((_CS, _C), lambda b, i, *_: (0, 0)),
            pl.BlockSpec((25, _C), lambda b, i, *_: (0, 0)),
            pl.BlockSpec((4, _C), lambda b, i, *_: (0, 0)),
            pl.BlockSpec((1, 4, _C), lambda b, i, *_: (b, 0, 0)),
            pl.BlockSpec((1, _C, 4), lambda b, i, *_: (b, 0, 0)),
        ],
        out_specs=pl.BlockSpec((1, _C, _R, 224), lambda b, i, *_: (b, 0, i, 0)),
        scratch_shapes=[
            pltpu.VMEM((_R + 8, 256, 128), jnp.float32),
            pltpu.VMEM((_RH, _CW, _KP), jnp.bfloat16),
            pltpu.VMEM((_RH, _CW, _CS), jnp.float32),
            pltpu.VMEM((_RH, 114, _C), jnp.float32),
            pltpu.VMEM((_RH, 114, _C), jnp.float32),
            pltpu.VMEM((_R, 224, _CS), jnp.float32),
            pltpu.VMEM((_RH, _CW, _CS), jnp.bfloat16),
            pltpu.VMEM((_R, 224, _C), jnp.float32),
        ],
    )
    out = pl.pallas_call(
        _body,
        grid_spec=spec,
        out_shape=jax.ShapeDtypeStruct((_B, _C, _H, _W), f32),
    )(selflags, xp, eye, wcat0, wcat1, dwW, pwW, g2w, biasr, wcomb, wcombT)
    return out


# speedup vs baseline: 7.8517x; 1.0001x over previous
import functools

import jax
import jax.numpy as jnp
from jax.experimental import pallas as pl
from jax.experimental.pallas import tpu as pltpu

_B, _C, _H, _W = 2, 96, 224, 224
_CS = _C // 2
_R = 16
_NB = _H // _R
_RH = _R + 4
_KP = 9 * 128
_CW = 228


def _pad_body(x_ref, o_ref):
    o_ref[...] = jnp.zeros(o_ref.shape, jnp.bfloat16)
    o_ref[0, :, 3:227, 16:240] = x_ref[0].astype(jnp.bfloat16)


def _pad_cast(x):
    return pl.pallas_call(
        _pad_body,
        grid=(_B, 6),
        in_specs=[pl.BlockSpec((1, 16, 224, 224), lambda b, c: (b, c, 0, 0))],
        out_specs=pl.BlockSpec((1, 16, 232, 256), lambda b, c: (b, c, 0, 0)),
        out_shape=jax.ShapeDtypeStruct((_B, _C, 232, 256), jnp.bfloat16),
    )(x)


def _body(sel_ref, x_ref, eye_ref, wc0_ref, wc1_ref, dw_ref, pw_ref, g2_ref,
          bias_ref, wc_ref, wct_ref, o_ref, xs_ref, sc_ref, ya_ref, yp_ref,
          ysw_ref, y2u_ref, xir_ref, acc_ref):
    b = pl.program_id(0)
    i = pl.program_id(1)
    s0 = sel_ref[b, 0]
    s1 = sel_ref[b, 1]
    s2 = sel_ref[b, 2]
    wc = wc_ref[0]

    @pl.when((b == 0) & (i == 0))
    def _init():
        xs_ref[...] = jnp.zeros((_R + 8, 256, 128), jnp.float32)
        sc_ref[...] = jnp.zeros((_RH, _CW, _KP), jnp.bfloat16)

    @pl.when(s0 + s1 + s2 > 0)
    def _experts():
        x2 = x_ref[0].reshape(_C, (_R + 8) * 256)
        xt = jax.lax.dot_general(x2, eye_ref[...], (((0,), (0,)), ((), ())),
                                 preferred_element_type=jnp.float32)
        xs_ref[:, :, 0:_C] = xt.reshape(_R + 8, 256, _C)

        xc = xs_ref[3:3 + _R, 16:240, 0:_C]
        acc_ref[...] = xc * wc[3:4, :].reshape(1, 1, _C)

        @pl.when(s0 + s2 > 0)
        def _e02():
            for t in range(9):
                di, dj = t // 3, t % 3
                sc_ref[:, :, 128 * t:128 * (t + 1)] = (
                    xs_ref[di:di + _RH, 13 + dj:13 + dj + _CW, :].astype(jnp.bfloat16))
            @pl.when(s0 > 0)
            def _e0():
                mme = jnp.dot(sc_ref[...].reshape(_RH * _CW, _KP), wc0_ref[...],
                              preferred_element_type=jnp.float32
                              ).reshape(_RH, _CW, _C)
                e0l = mme[2:2 + _R, 2:226, :] + bias_ref[0:1, :].reshape(1, 1, _C)
                acc_ref[...] += (e0l * jax.nn.sigmoid(e0l)) * wc[0:1, :].reshape(1, 1, _C)

            @pl.when(s2 > 0)
            def _e2():
                mmc = jnp.dot(sc_ref[...].reshape(_RH * _CW, _KP), wc1_ref[...],
                              preferred_element_type=jnp.float32
                              ).reshape(_RH, _CW, _CS)
                yl = mmc + bias_ref[2:3, 0:_CS].reshape(1, 1, _CS)
                ya_ref[...] = yl * jax.nn.sigmoid(yl)
                yp_ref[:, :, 0:_CS] = ya_ref[:, 0:_CW:2, :]
                yp_ref[:, :, _CS:_C] = ya_ref[:, 1:_CW:2, :]
                ysw_ref[:, 0:113, 0:_CS] = yp_ref[:, 0:113, _CS:_C]
                ysw_ref[:, 0:113, _CS:_C] = yp_ref[:, 1:114, 0:_CS]
                y2p = jnp.broadcast_to(
                    bias_ref[3:4, :].reshape(1, 1, _C), (_R, 112, _C)
                ).astype(jnp.float32)
                for u in range(5):
                    y2p += yp_ref[u:u + _R, 0:112, :] * g2_ref[5 * u:5 * u + 1, :].reshape(1, 1, _C)
                    y2p += ysw_ref[u:u + _R, 0:112, :] * g2_ref[5 * u + 1:5 * u + 2, :].reshape(1, 1, _C)
                    y2p += yp_ref[u:u + _R, 1:113, :] * g2_ref[5 * u + 2:5 * u + 3, :].reshape(1, 1, _C)
                    y2p += ysw_ref[u:u + _R, 1:113, :] * g2_ref[5 * u + 3:5 * u + 4, :].reshape(1, 1, _C)
                    y2p += yp_ref[u:u + _R, 2:114, :] * g2_ref[5 * u + 4:5 * u + 5, :].reshape(1, 1, _C)
                y2a = y2p * jax.nn.sigmoid(y2p)
                y2u_ref[:, 0:224:2, :] = y2a[:, :, 0:_CS]
                y2u_ref[:, 1:224:2, :] = y2a[:, :, _CS:_C]
                e2 = jnp.concatenate([ya_ref[2:2 + _R, 2:226, :], y2u_ref[...]],
                                     axis=-1)
                acc_ref[...] += e2 * wc[2:3, :].reshape(1, 1, _C)

        @pl.when(s1 > 0)
        def _e1():
            xir_ref[...] = xs_ref[1:1 + _RH, 14:242, _CS:_C]
            z = jnp.zeros((_R, 224, _CS), jnp.float32)
            for t in range(25):
                u, v = t // 5, t % 5
                z += (xir_ref[u:u + _R, v:v + 224, :]
                      * dw_ref[t:t + 1, :].reshape(1, 1, _CS))
            p = jnp.dot(z.astype(jnp.bfloat16).reshape(_R * 224, _CS), pw_ref[...],
                        preferred_element_type=jnp.float32)
            e1l = p.reshape(_R, 224, _C) + bias_ref[1:2, :].reshape(1, 1, _C)
            acc_ref[...] += (e1l * jax.nn.sigmoid(e1l)) * wc[1:2, :].reshape(1, 1, _C)

        o_ref[0] = jnp.transpose(acc_ref[...], (2, 0, 1))

    @pl.when(s0 + s1 + s2 == 0)
    def _identity_only():
        w3o = wct_ref[0, :, 3:4].reshape(_C, 1, 1)
        o_ref[0] = x_ref[0, :, 3:3 + _R, 16:240].astype(jnp.float32) * w3o


@functools.partial(jax.jit, static_argnums=())
def kernel(x, weights, indices, W0, g0, b0, m0, v0, Wdw, Wpw, g1, b1, m1, v1,
           Wg1, gg1, bg1, mg1, vg1, Wg2, gg2, bg2, mg2, vg2):
    f32 = jnp.float32
    eps = 1e-5

    s0 = g0 / jnp.sqrt(v0 + eps); t0 = b0 - m0 * s0
    W0f = W0 * s0[:, None, None, None]
    s1 = g1 / jnp.sqrt(v1 + eps); t1 = b1 - m1 * s1
    Wpwf = Wpw * s1[:, None, None, None]
    sg1 = gg1 / jnp.sqrt(vg1 + eps); tg1 = bg1 - mg1 * sg1
    Wg1f = Wg1 * sg1[:, None, None, None]
    sg2 = gg2 / jnp.sqrt(vg2 + eps); tg2 = bg2 - mg2 * sg2
    Wg2f = Wg2 * sg2[:, None, None, None]

    A0 = jnp.pad(W0f.transpose(2, 3, 1, 0), ((0, 0), (0, 0), (0, 128 - _CS), (0, 0)))
    A0 = A0.reshape(_KP, _C)
    A1 = jnp.pad(Wg1f.transpose(2, 3, 1, 0), ((0, 0), (0, 0), (0, 128 - _C), (0, 0)))
    A1 = A1.reshape(_KP, _CS)
    wcat0 = A0.astype(jnp.bfloat16)
    wcat1 = A1.astype(jnp.bfloat16)

    dwW = Wdw[:, 0].transpose(1, 2, 0).reshape(25, _CS)
    pwW = Wpwf[:, :, 0, 0].T.astype(jnp.bfloat16)
    g2w = jnp.tile(Wg2f[:, 0].transpose(1, 2, 0).reshape(25, _CS), (1, 2))
    biasr = jnp.stack([t0, t1, jnp.pad(tg1, (0, _CS)),
                       jnp.concatenate([tg2, tg2])])

    onehot = (indices[:, :, None] == jnp.arange(4)[None, None, :]).astype(f32)
    wsum = jnp.einsum('bke,bkc->bec', onehot, weights)
    idc = 0.1 * jnp.sum(onehot[:, :, 3], axis=1)
    wcomb = wsum.at[:, 3, :].set(idc[:, None])
    wcombT = jnp.transpose(wcomb, (0, 2, 1))
    selflags = (jnp.sum(onehot, axis=1) > 0).astype(jnp.int32)

    xp = _pad_cast(x)
    eye = jnp.eye(_C, dtype=jnp.bfloat16)

    spec = pltpu.PrefetchScalarGridSpec(
        num_scalar_prefetch=1,
        grid=(_B, _NB),
        in_specs=[
            pl.BlockSpec((pl.Element(1), pl.Element(_C), pl.Element(_R + 8),
                          pl.Element(256)), lambda b, i, *_: (b, 0, i * _R, 0)),
            pl.BlockSpec((_C, _C), lambda b, i, *_: (0, 0)),
            pl.BlockSpec((_KP, _C), lambda b, i, *_: (0, 0)),
            pl.BlockSpec((_KP, _CS), lambda b, i, *_: (0, 0)),
            pl.BlockSpec((25, _CS), lambda b, i, *_: (0, 0)),
            pl.BlockSpec((_CS, _C), lambda b, i, *_: (0, 0)),
            pl.BlockSpec((25, _C), lambda b, i, *_: (0, 0)),
            pl.BlockSpec((4, _C), lambda b, i, *_: (0, 0)),
            pl.BlockSpec((1, 4, _C), lambda b, i, *_: (b, 0, 0)),
            pl.BlockSpec((1, _C, 4), lambda b, i, *_: (b, 0, 0)),
        ],
        out_specs=pl.BlockSpec((1, _C, _R, 224), lambda b, i, *_: (b, 0, i, 0)),
        scratch_shapes=[
            pltpu.VMEM((_R + 8, 256, 128), jnp.float32),
            pltpu.VMEM((_RH, _CW, _KP), jnp.bfloat16),
            pltpu.VMEM((_RH, _CW, _CS), jnp.float32),
            pltpu.VMEM((_RH, 114, _C), jnp.float32),
            pltpu.VMEM((_RH, 114, _C), jnp.float32),
            pltpu.VMEM((_R, 224, _CS), jnp.float32),
            pltpu.VMEM((_RH, _CW, _CS), jnp.float32),
            pltpu.VMEM((_R, 224, _C), jnp.float32),
        ],
    )
    out = pl.pallas_call(
        _body,
        grid_spec=spec,
        out_shape=jax.ShapeDtypeStruct((_B, _C, _H, _W), f32),
    )(selflags, xp, eye, wcat0, wcat1, dwW, pwW, g2w, biasr, wcomb, wcombT)
    return out
